# Initial kernel scaffold; baseline (speedup 1.0000x reference)
import jax
import jax.numpy as jnp
from jax.experimental import pallas as pl


def kernel(x, W, b, edge_index, actual_amount):
    raise NotImplementedError("write your pallas kernel here")


# speedup vs baseline: 46.2659x; 46.2659x over previous
import functools

import jax
import jax.numpy as jnp
from jax import lax
from jax.experimental import pallas as pl
from jax.experimental.pallas import tpu as pltpu
from jax.experimental.pallas import tpu_sc as plsc

NODE_DIM = 128
N_NODES = 10000
N_EDGES = 320000

NW = 32
EPT = N_EDGES // NW
VPT = EPT // 16


def _proj_body(x_ref, wc_ref, bias_ref, o_ref):
    o_ref[...] = lax.dot_general(
        x_ref[...], wc_ref[...], (((1,), (1,)), ((), ())),
        preferred_element_type=jnp.float32,
    ) + bias_ref[...]


def _projection(x, wc, bias):
    rows_blk = 2000
    return pl.pallas_call(
        _proj_body,
        grid=(N_NODES // rows_blk,),
        in_specs=[
            pl.BlockSpec((rows_blk, NODE_DIM), lambda i: (i, 0)),
            pl.BlockSpec((4, NODE_DIM), lambda i: (0, 0)),
            pl.BlockSpec((1, 4), lambda i: (0, 0)),
        ],
        out_specs=pl.BlockSpec((rows_blk, 4), lambda i: (i, 0)),
        out_shape=jax.ShapeDtypeStruct((N_NODES, 4), jnp.float32),
    )(x, wc, bias)


def _edge_pass1(tab_hbm, row_hbm, col_hbm,
                ex0_hbm, ex1_hbm, part_hbm,
                tab_v, row_v, col_v, ex0_v, ex1_v, den_v):
    wid = lax.axis_index("s") * 2 + lax.axis_index("c")
    base = wid * EPT
    pltpu.sync_copy(tab_hbm, tab_v)
    pltpu.sync_copy(row_hbm.at[pl.ds(base, EPT)], row_v)
    pltpu.sync_copy(col_hbm.at[pl.ds(base, EPT)], col_v)

    zeros = jnp.zeros((16,), jnp.float32)

    def _zero(i, _):
        den_v[pl.ds(i * 16, 16)] = zeros
        return ()

    lax.fori_loop(0, 2 * N_NODES // 16, _zero, ())

    def _step(i, _):
        sl = pl.ds(i * 16, 16)
        r = row_v[sl]
        cl = col_v[sl]
        r4 = r * 4
        c4 = cl * 4
        a0 = plsc.load_gather(tab_v, [r4])
        a1 = plsc.load_gather(tab_v, [r4 + 1])
        c0 = plsc.load_gather(tab_v, [c4 + 2])
        c1 = plsc.load_gather(tab_v, [c4 + 3])
        s0 = jnp.minimum(jnp.maximum(a0 + c0, 0.0), 80.0)
        s1 = jnp.minimum(jnp.maximum(a1 + c1, 0.0), 80.0)
        e0 = jnp.exp(s0)
        e1 = jnp.exp(s1)
        ex0_v[sl] = e0
        ex1_v[sl] = e1
        r2 = r * 2
        plsc.addupdate_scatter(den_v, [r2], e0)
        plsc.addupdate_scatter(den_v, [r2 + 1], e1)
        return ()

    lax.fori_loop(0, VPT, _step, ())

    pltpu.sync_copy(ex0_v, ex0_hbm.at[pl.ds(base, EPT)])
    pltpu.sync_copy(ex1_v, ex1_hbm.at[pl.ds(base, EPT)])
    pltpu.sync_copy(den_v, part_hbm.at[wid])


def _reduce_body(p_ref, o_ref):
    o_ref[...] = jnp.sum(p_ref[...], axis=0, keepdims=True)


def _reduce_partials(part):
    return pl.pallas_call(
        _reduce_body,
        out_shape=jax.ShapeDtypeStruct((1, 2 * N_NODES), jnp.float32),
    )(part)


def _edge_pass2(den_hbm, row_hbm, ex0_hbm, ex1_hbm, amt_hbm,
                out_hbm,
                den_v, row_v, ex0_v, ex1_v, amt_v, out_v):
    wid = lax.axis_index("s") * 2 + lax.axis_index("c")
    base = wid * EPT
    pltpu.sync_copy(den_hbm, den_v)
    pltpu.sync_copy(row_hbm.at[pl.ds(base, EPT)], row_v)
    pltpu.sync_copy(ex0_hbm.at[pl.ds(base, EPT)], ex0_v)
    pltpu.sync_copy(ex1_hbm.at[pl.ds(base, EPT)], ex1_v)
    pltpu.sync_copy(amt_hbm.at[pl.ds(base, EPT)], amt_v)

    eps = jnp.float32(1e-16)

    def _step(i, _):
        sl = pl.ds(i * 16, 16)
        r2 = row_v[sl] * 2
        d0 = plsc.load_gather(den_v, [r2])
        d1 = plsc.load_gather(den_v, [r2 + 1])
        val = ex0_v[sl] / (d0 + eps) + ex1_v[sl] / (d1 + eps)
        val = jnp.where(amt_v[sl] != 0, val, 0.0)
        out_v[sl] = val
        return ()

    lax.fori_loop(0, VPT, _step, ())
    pltpu.sync_copy(out_v, out_hbm.at[pl.ds(base, EPT)])


def _sc_mesh():
    return plsc.VectorSubcoreMesh(core_axis_name="c", subcore_axis_name="s")


@functools.partial(
    pl.kernel,
    out_type=(
        jax.ShapeDtypeStruct((N_EDGES,), jnp.float32),
        jax.ShapeDtypeStruct((N_EDGES,), jnp.float32),
        jax.ShapeDtypeStruct((NW, 2 * N_NODES), jnp.float32),
    ),
    mesh=_sc_mesh(),
    scratch_types=[
        pltpu.VMEM((4 * N_NODES,), jnp.float32),
        pltpu.VMEM((EPT,), jnp.int32),
        pltpu.VMEM((EPT,), jnp.int32),
        pltpu.VMEM((EPT,), jnp.float32),
        pltpu.VMEM((EPT,), jnp.float32),
        pltpu.VMEM((2 * N_NODES,), jnp.float32),
    ],
    compiler_params=pltpu.CompilerParams(needs_layout_passes=False),
)
def _sc_pass1(*refs):
    _edge_pass1(*refs)


@functools.partial(
    pl.kernel,
    out_type=jax.ShapeDtypeStruct((N_EDGES,), jnp.float32),
    mesh=_sc_mesh(),
    scratch_types=[
        pltpu.VMEM((2 * N_NODES,), jnp.float32),
        pltpu.VMEM((EPT,), jnp.int32),
        pltpu.VMEM((EPT,), jnp.float32),
        pltpu.VMEM((EPT,), jnp.float32),
        pltpu.VMEM((EPT,), jnp.int32),
        pltpu.VMEM((EPT,), jnp.float32),
    ],
    compiler_params=pltpu.CompilerParams(needs_layout_passes=False),
)
def _sc_pass2(*refs):
    _edge_pass2(*refs)


def kernel(x, W, b, edge_index, actual_amount):
    row = edge_index[0].astype(jnp.int32)
    col = edge_index[1].astype(jnp.int32)
    amt = actual_amount.astype(jnp.int32)

    wc = W.reshape(2, 2, NODE_DIM).transpose(1, 0, 2).reshape(4, NODE_DIM)
    bias = jnp.concatenate([b, jnp.zeros((2,), jnp.float32)]).reshape(1, 4)

    tab = _projection(x, wc, bias).reshape(-1)

    ex0, ex1, part = _sc_pass1(tab, row, col)
    den = _reduce_partials(part).reshape(-1)
    out = _sc_pass2(den, row, ex0, ex1, amt)
    return out
